# baseline (speedup 1.0000x reference)
import functools

import jax
import jax.numpy as jnp
from jax import lax
from jax.experimental import pallas as pl
from jax.experimental.pallas import tpu as pltpu
from jax.experimental.pallas import tpu_sc as plsc

NUM_NODES = 1000000
EMBED_DIM = 64
BATCH = 16384

_info = plsc.get_sparse_core_info()
_NC, _NS = _info.num_cores, _info.num_subcores
_NW = _NC * _NS
_BLK = 128
_NCOL = (NUM_NODES + _BLK - 1) // _BLK
_CPW = 256
_CAND = BATCH + 16

_mesh = plsc.VectorSubcoreMesh(core_axis_name="c", subcore_axis_name="s")

@functools.partial(
    pl.kernel,
    mesh=_mesh,
    out_type=jax.ShapeDtypeStruct((_CAND, _BLK), jnp.float32),
    scratch_types=[
        pltpu.VMEM((BATCH,), jnp.int32),
        pltpu.VMEM((BATCH,), jnp.int32),
        pltpu.VMEM((BATCH,), jnp.int32),
        pltpu.VMEM((BATCH,), jnp.int32),
        pltpu.VMEM((_CPW,), jnp.int32),
        pltpu.VMEM((_CPW,), jnp.int32),
        *[pltpu.VMEM((EMBED_DIM, _BLK), jnp.float32) for _ in range(6)],
        pltpu.VMEM((16, _BLK), jnp.float32),
        pltpu.VMEM((16,), jnp.int32),
        *[pltpu.SemaphoreType.DMA for _ in range(7)],
    ],
    compiler_params=pltpu.CompilerParams(needs_layout_passes=False),
)
def _dedup_kernel(idx_hbm, tableT_hbm, cand_hbm, idxg, lr_v, lj_v, qj_v,
                  bmap, clist, t0, t1, t2, t3, t4, t5, mrows, midx,
                  s0, s1, s2, s3, s4, s5, semw):
    qr_v = idxg
    w = lax.axis_index("s") * _NC + lax.axis_index("c")
    lane = lax.iota(jnp.int32, 16)
    ones = lane * 0 + 1
    trash = lane * 0 + BATCH
    c16 = [lane + 16 * k for k in range(EMBED_DIM // 16)]
    tbufs = (t0, t1, t2, t3, t4, t5)
    sems = (s0, s1, s2, s3, s4, s5)

    pltpu.sync_copy(idx_hbm, idxg)
    for q in range(_CPW // 16):
        bmap[pl.ds(16 * q, 16)] = lane * 0
    midx[pl.ds(0, 16)] = trash

    def scan_body(i, nm):
        v = idxg[pl.ds(16 * i, 16)]
        col = lax.shift_right_logical(v, 7)
        mine = lax.shift_right_logical(col, 8) == w
        plsc.store_scatter(bmap, [col & (_CPW - 1)], ones, mask=mine)
        pos = plsc.cumsum(jnp.where(mine, 1, 0))
        plsc.store_scatter(lr_v, [nm + pos - 1], v, mask=mine)
        plsc.store_scatter(lj_v, [nm + pos - 1], lane + 16 * i, mask=mine)
        return nm + plsc.all_reduce_population_count(mine)[0]

    nm = lax.fori_loop(0, BATCH // 16, scan_body, jnp.int32(0))

    def cmp_body(q, nc):
        f = bmap[pl.ds(16 * q, 16)] > 0
        pos = plsc.cumsum(jnp.where(f, 1, 0))
        plsc.store_scatter(clist, [nc + pos - 1], lane + 16 * q, mask=f)
        return nc + plsc.all_reduce_population_count(f)[0]

    nc = lax.fori_loop(0, _CPW // 16, cmp_body, jnp.int32(0))

    def at_scalar(ref, p):
        vec = ref[pl.ds(lax.shift_right_logical(p, 4) * 16, 16)]
        return jnp.sum(jnp.where(lane == (p & 15), vec, 0))

    def fetch(colg, b):
        rbase = pl.multiple_of(colg * _BLK, _BLK)
        pltpu.async_copy(tableT_hbm.at[:, pl.ds(rbase, _BLK)],
                         tbufs[b], sems[b])

    def wait_fetch(b):
        pltpu.make_async_copy(tableT_hbm.at[:, pl.ds(0, _BLK)],
                              tbufs[b], sems[b]).wait()

    def flush():
        pltpu.async_copy(mrows, cand_hbm.at[midx], semw)
        pltpu.make_async_copy(mrows, cand_hbm.at[midx], semw).wait()
        midx[pl.ds(0, 16)] = trash

    for d in range(5):
        @pl.when(d < nc)
        def _():
            fetch(at_scalar(clist, jnp.int32(d)) + _CPW * w, d)

    def col_body(k, mtot):
        for b in range(6):
            i = 6 * k + b

            def process(mtot):
                colg = at_scalar(clist, i) + _CPW * w
                wait_fetch(b)

                @pl.when(i + 5 < nc)
                def _():
                    fetch(at_scalar(clist, i + 5) + _CPW * w, (b + 5) % 6)

                def match_body(m, qn):
                    rv = lr_v[pl.ds(16 * m, 16)]
                    jv = lj_v[pl.ds(16 * m, 16)]
                    inb = (lane + 16 * m) < nm
                    mm = (lax.shift_right_logical(rv, 7) == colg) & inb
                    pos = plsc.cumsum(jnp.where(mm, 1, 0))
                    plsc.store_scatter(qr_v, [qn + pos - 1], rv, mask=mm)
                    plsc.store_scatter(qj_v, [qn + pos - 1], jv, mask=mm)
                    return qn + plsc.all_reduce_population_count(mm)[0]

                qn = lax.fori_loop(0, (nm + 15) // 16, match_body,
                                   jnp.int32(0))

                flat = tbufs[b].reshape(1, EMBED_DIM * _BLK).at[0]

                def drain_body(eb, mtot):
                    rv = qr_v[pl.ds(16 * eb, 16)] & (_BLK - 1)
                    jv = qj_v[pl.ds(16 * eb, 16)]
                    for u in range(16):
                        @pl.when(16 * eb + u < qn)
                        def _():
                            slot = (mtot + u) & 15
                            rr = (lane & 0) + rv[u]
                            for kk in range(EMBED_DIM // 16):
                                vals = plsc.load_gather(
                                    flat, [c16[kk] * _BLK + rr])
                                mrows[slot, pl.ds(16 * kk, 16)] = vals
                            plsc.store_scatter(midx, [(lane & 0) + slot],
                                               (lane & 0) + jv[u],
                                               mask=lane == 0)

                            @pl.when(slot == 15)
                            def _():
                                flush()
                    return mtot + jnp.minimum(qn - 16 * eb, 16)

                return lax.fori_loop(0, (qn + 15) >> 4, drain_body, mtot)

            mtot = lax.cond(i < nc, process, lambda m: m, mtot)
        return mtot

    mtot = lax.fori_loop(0, (_CPW + 5) // 6, col_body, jnp.int32(0))

    @pl.when((mtot & 15) != 0)
    def _():
        flush()


def kernel(nodes, ent_features):
    cand = _dedup_kernel(nodes.astype(jnp.int32), ent_features.T)
    return cand[:BATCH, :EMBED_DIM]


# speedup vs baseline: 1.6911x; 1.6911x over previous
import functools

import jax
import jax.numpy as jnp
from jax import lax
from jax.experimental import pallas as pl
from jax.experimental.pallas import tpu as pltpu
from jax.experimental.pallas import tpu_sc as plsc

NUM_NODES = 1000000
EMBED_DIM = 64
BATCH = 16384

_info = plsc.get_sparse_core_info()
_NC, _NS = _info.num_cores, _info.num_subcores
_NW = _NC * _NS
_B_PER_W = BATCH // _NW
_BLK = 128

_mesh = plsc.VectorSubcoreMesh(core_axis_name="c", subcore_axis_name="s")


@functools.partial(
    pl.kernel,
    mesh=_mesh,
    out_type=jax.ShapeDtypeStruct((BATCH, EMBED_DIM), jnp.float32),
    scratch_types=[
        pltpu.VMEM((_B_PER_W,), jnp.int32),
        *[pltpu.VMEM((EMBED_DIM, _BLK), jnp.float32) for _ in range(8)],
        pltpu.VMEM((_B_PER_W // 2, EMBED_DIM), jnp.float32),
        *[pltpu.SemaphoreType.DMA for _ in range(8)],
        pltpu.SemaphoreType.DMA,
    ],
    compiler_params=pltpu.CompilerParams(needs_layout_passes=False),
)
def _gather_kernel(idx_hbm, tableT_hbm, out_hbm, idx_v,
                   t0, t1, t2, t3, t4, t5, t6, t7, rows_v,
                   sem0, sem1, sem2, sem3, sem4, sem5, sem6, sem7, semw):
    wid = lax.axis_index("s") * _NC + lax.axis_index("c")
    base = wid * _B_PER_W
    pltpu.sync_copy(idx_hbm.at[pl.ds(base, _B_PER_W)], idx_v)

    tbufs = (t0, t1, t2, t3, t4, t5, t6, t7)
    sems = (sem0, sem1, sem2, sem3, sem4, sem5, sem6, sem7)
    c16 = [lax.iota(jnp.int32, 16) + 16 * k for k in range(EMBED_DIM // 16)]

    def start_fetch(r, b):
        rbase = pl.multiple_of(r & ~(_BLK - 1), _BLK)
        pltpu.async_copy(tableT_hbm.at[:, pl.ds(rbase, _BLK)],
                         tbufs[b], sems[b])

    def wait_fetch(b):
        pltpu.make_async_copy(tableT_hbm.at[:, pl.ds(0, _BLK)],
                              tbufs[b], sems[b]).wait()

    def extract(j, r, b):
        rr = (c16[0] & 0) + (r & (_BLK - 1))
        for k in range(EMBED_DIM // 16):
            flat = tbufs[b].reshape(1, EMBED_DIM * _BLK).at[0]
            vals = plsc.load_gather(flat, [c16[k] * _BLK + rr])
            rows_v[j, pl.ds(16 * k, 16)] = vals

    _DEPTH = 7
    _NG = _B_PER_W // 16
    _HG = _NG // 2

    v0 = idx_v[pl.ds(0, 16)]
    for d in range(_DEPTH):
        start_fetch(v0[d], d)

    def make_body(half):
        def body(g, carry):
            j0 = 16 * (half * _HG + g)
            v = idx_v[pl.ds(j0, 16)]
            for u in range(16):
                r = v[u]
                b = u & 7
                wait_fetch(b)
                nb = (u + _DEPTH) & 7
                if u < 16 - _DEPTH:
                    start_fetch(v[u + _DEPTH], nb)
                else:
                    @pl.when(j0 + 16 + (u - (16 - _DEPTH)) < _B_PER_W)
                    def _():
                        vn = idx_v[pl.ds(j0 + 16, 16)]
                        start_fetch(vn[u - (16 - _DEPTH)], nb)
                extract(16 * g + u, r, b)
            return carry
        return body

    lax.fori_loop(0, _HG, make_body(0), 0)
    pltpu.sync_copy(rows_v, out_hbm.at[pl.ds(base, _B_PER_W // 2)])
    lax.fori_loop(0, _HG, make_body(1), 0)
    pltpu.sync_copy(rows_v,
                    out_hbm.at[pl.ds(base + _B_PER_W // 2, _B_PER_W // 2)])


def kernel(nodes, ent_features):
    return _gather_kernel(nodes.astype(jnp.int32), ent_features.T)
